# baseline (speedup 1.0000x reference)
import jax
import jax.numpy as jnp
from jax import lax
from jax.experimental import pallas as pl
from jax.experimental.pallas import tpu as pltpu

VOCAB_SIZE = 100
NUM_IDS = 327680
ROWS_PER_BLOCK = 4096
NUM_BLOCKS = NUM_IDS // ROWS_PER_BLOCK


def _onehot_block(ids_ref, out_ref):
    ids = ids_ref[0]
    cols = lax.broadcasted_iota(jnp.int32, (ROWS_PER_BLOCK, VOCAB_SIZE), 1)
    out_ref[...] = (ids == cols).astype(jnp.int32)


def kernel(input):
    ids3 = input.reshape(NUM_BLOCKS, ROWS_PER_BLOCK, 1)
    out = pl.pallas_call(
        _onehot_block,
        grid=(NUM_BLOCKS,),
        in_specs=[pl.BlockSpec((1, ROWS_PER_BLOCK, 1), lambda i: (i, 0, 0))],
        out_specs=pl.BlockSpec((ROWS_PER_BLOCK, VOCAB_SIZE), lambda i: (i, 0)),
        out_shape=jax.ShapeDtypeStruct((NUM_IDS, VOCAB_SIZE), jnp.int32),
        compiler_params=pltpu.CompilerParams(
            dimension_semantics=("arbitrary",),
        ),
    )(ids3)
    return out
